# baseline (speedup 1.0000x reference)
import functools

import jax
import jax.numpy as jnp
from jax import lax
from jax.experimental import pallas as pl
from jax.experimental.pallas import tpu as pltpu
from jax.experimental.pallas import tpu_sc as plsc

_NC = 2
_NS = 16
_NW = _NC * _NS
_LANES = 16
_CHUNK = 128


@functools.cache
def _build(B, S, D):
    N = B * S
    T = N // _NW
    n_chunks = T // _CHUNK
    mesh = plsc.VectorSubcoreMesh(
        core_axis_name="c", subcore_axis_name="s",
        num_cores=_NC, num_subcores=_NS)

    @functools.partial(
        pl.kernel,
        out_type=jax.ShapeDtypeStruct((N, D), jnp.float32),
        mesh=mesh,
        scratch_types=[
            pltpu.VMEM((n_chunks, _CHUNK), jnp.int32),
            pltpu.VMEM((_CHUNK, D), jnp.float32),
            pltpu.VMEM((T, D), jnp.float32),
            pltpu.SemaphoreType.DMA,
        ],
        compiler_params=pltpu.CompilerParams(use_tc_tiling_on_sc=False),
    )
    def _k(ids_hbm, tok_hbm, pos_hbm, out_hbm, idx_v, rows_v, pos_v, sem):
        wid = lax.axis_index("s") * _NC + lax.axis_index("c")
        tok0 = wid * T
        s0 = lax.rem(tok0, S)
        pltpu.sync_copy(ids_hbm.at[pl.ds(wid * n_chunks, n_chunks)], idx_v)
        pltpu.sync_copy(pos_hbm.at[pl.ds(s0, T)], pos_v)
        for j in range(n_chunks):
            pltpu.async_copy(tok_hbm.at[idx_v.at[j]], rows_v, sem).wait()

            def add_row(i, _):
                for q in range(D // _LANES):
                    sl = pl.ds(q * _LANES, _LANES)
                    rows_v[i, sl] += pos_v[j * _CHUNK + i, sl]
                return 0

            lax.fori_loop(0, _CHUNK, add_row, 0)
            pltpu.sync_copy(
                rows_v, out_hbm.at[pl.ds(tok0 + j * _CHUNK, _CHUNK)])

    return _k


def kernel(input_ids, token_table, pos_table):
    B, S = input_ids.shape
    D = token_table.shape[1]
    ids = input_ids.reshape(-1).astype(jnp.int32).reshape(-1, _CHUNK)
    out = _build(B, S, D)(ids, token_table, pos_table)
    return out.reshape(B, S, D)
